# Initial kernel scaffold; baseline (speedup 1.0000x reference)
import jax
import jax.numpy as jnp
from jax.experimental import pallas as pl


def kernel(x_graph, edge_index, W1, b1, W2, b2):
    raise NotImplementedError("write your pallas kernel here")


# speedup vs baseline: 32.6583x; 32.6583x over previous
import functools

import jax
import jax.numpy as jnp
from jax import lax
from jax.experimental import pallas as pl
from jax.experimental.pallas import tpu as pltpu
from jax.experimental.pallas import tpu_sc as plsc

N = 10000
D_IN = 128
D_HID = 16
D_OUT = 128

NC = 2
NS = 16
NW = NC * NS
CHUNK = 128
ACC_ROWS = 10240
RPT = ACC_ROWS // NS

_MESH = dict(core_axis_name="c", subcore_axis_name="s")


def _pad_edges(edge_index):
    e = edge_index.shape[1]
    n_chunks = -(-e // (NW * CHUNK))
    epad = NW * n_chunks * CHUNK
    src = jnp.concatenate(
        [edge_index[0], jnp.zeros((epad - e,), jnp.int32)])
    dst = jnp.concatenate(
        [edge_index[1], jnp.full((epad - e,), N, jnp.int32)])
    return (src.reshape(NW, n_chunks, CHUNK),
            dst.reshape(NW, n_chunks, CHUNK), n_chunks)


def _sc_deg(dst3, n_chunks):
    @functools.partial(
        pl.kernel,
        out_type=jax.ShapeDtypeStruct((NC, ACC_ROWS), jnp.float32),
        mesh=plsc.VectorSubcoreMesh(**_MESH),
        scratch_types=[
            pltpu.VMEM_SHARED((ACC_ROWS,), jnp.float32),
            pltpu.VMEM((n_chunks, CHUNK), jnp.int32),
            pltpu.VMEM((CHUNK,), jnp.float32),
            pltpu.VMEM((RPT,), jnp.float32),
        ],
    )
    def k(dst_hbm, out_hbm, acc_sh, idx_v, ones_v, stage_v):
        c = lax.axis_index("c")
        s = lax.axis_index("s")
        wid = c * NS + s

        def fill_zero(i, carry):
            stage_v[pl.ds(i * 16, 16)] = jnp.zeros((16,), jnp.float32)
            return carry

        lax.fori_loop(0, RPT // 16, fill_zero, 0)

        def fill_one(i, carry):
            ones_v[pl.ds(i * 16, 16)] = jnp.ones((16,), jnp.float32)
            return carry

        lax.fori_loop(0, CHUNK // 16, fill_one, 0)

        pltpu.sync_copy(stage_v, acc_sh.at[pl.ds(s * RPT, RPT)])
        pltpu.sync_copy(dst_hbm.at[wid], idx_v)
        plsc.subcore_barrier()

        def body(j, carry):
            pltpu.sync_copy(ones_v, acc_sh.at[idx_v.at[j]], add=True)
            return carry

        lax.fori_loop(0, n_chunks, body, 0)
        plsc.subcore_barrier()
        pltpu.sync_copy(acc_sh.at[pl.ds(s * RPT, RPT)], stage_v)
        pltpu.sync_copy(stage_v, out_hbm.at[c, pl.ds(s * RPT, RPT)])

    return k(dst3)


def _sc_edge(u, src3, dst3, n_chunks):
    @functools.partial(
        pl.kernel,
        out_type=jax.ShapeDtypeStruct((NC, ACC_ROWS, D_HID), jnp.float32),
        mesh=plsc.VectorSubcoreMesh(**_MESH),
        scratch_types=[
            pltpu.VMEM_SHARED((ACC_ROWS, D_HID), jnp.float32),
            pltpu.VMEM((n_chunks, CHUNK), jnp.int32),
            pltpu.VMEM((n_chunks, CHUNK), jnp.int32),
            pltpu.VMEM((CHUNK, D_HID), jnp.float32),
            pltpu.VMEM((RPT, D_HID), jnp.float32),
            pltpu.SemaphoreType.DMA,
        ],
        compiler_params=pltpu.CompilerParams(use_tc_tiling_on_sc=False),
    )
    def k(u_hbm, src_hbm, dst_hbm, out_hbm,
          acc_sh, srcv, dstv, rows_v, stage_v, sem):
        c = lax.axis_index("c")
        s = lax.axis_index("s")
        wid = c * NS + s

        def fill_zero(i, carry):
            stage_v[i] = jnp.zeros((D_HID,), jnp.float32)
            return carry

        lax.fori_loop(0, RPT, fill_zero, 0)
        pltpu.sync_copy(stage_v, acc_sh.at[pl.ds(s * RPT, RPT)])
        pltpu.sync_copy(src_hbm.at[wid], srcv)
        pltpu.sync_copy(dst_hbm.at[wid], dstv)
        plsc.subcore_barrier()

        def body(j, carry):
            pltpu.async_copy(u_hbm.at[srcv.at[j]], rows_v, sem).wait()
            pltpu.sync_copy(rows_v, acc_sh.at[dstv.at[j]], add=True)
            return carry

        lax.fori_loop(0, n_chunks, body, 0)
        plsc.subcore_barrier()
        pltpu.sync_copy(acc_sh.at[pl.ds(s * RPT, RPT)], stage_v)
        pltpu.sync_copy(stage_v, out_hbm.at[c, pl.ds(s * RPT, RPT)])

    return k(u, src3, dst3)


def _tc_scale_in(x, w1, deg_part):
    def body(x_ref, w_ref, degp_ref, u_ref, dinv_ref):
        deg = degp_ref[0] + degp_ref[1] + 1.0
        dinv = lax.rsqrt(deg)
        p = jnp.dot(x_ref[...], w_ref[...], preferred_element_type=jnp.float32)
        u_ref[...] = p * dinv
        dinv_ref[...] = dinv

    return pl.pallas_call(
        body,
        out_shape=(jax.ShapeDtypeStruct((N, D_HID), jnp.float32),
                   jax.ShapeDtypeStruct((N, 1), jnp.float32)),
    )(x, w1, deg_part)


def _tc_mid(s1, u1, dinv, b1):
    def body(s_ref, u_ref, dinv_ref, b_ref, out_ref):
        agg = dinv_ref[...] * (s_ref[0] + s_ref[1] + u_ref[...])
        h = jnp.maximum(agg + b_ref[...], 0.0)
        out_ref[...] = h * dinv_ref[...]

    return pl.pallas_call(
        body,
        out_shape=jax.ShapeDtypeStruct((N, D_HID), jnp.float32),
    )(s1, u1, dinv, b1)


def _tc_out(s2, u2, dinv, w2, b2):
    def body(s_ref, u_ref, dinv_ref, w_ref, b_ref, out_ref):
        agg = dinv_ref[...] * (s_ref[0] + s_ref[1] + u_ref[...])
        out_ref[...] = (
            jnp.dot(agg, w_ref[...], preferred_element_type=jnp.float32)
            + b_ref[...])

    return pl.pallas_call(
        body,
        out_shape=jax.ShapeDtypeStruct((N, D_OUT), jnp.float32),
    )(s2, u2, dinv, w2, b2)


def kernel(x_graph, edge_index, W1, b1, W2, b2):
    src3, dst3, n_chunks = _pad_edges(edge_index)

    deg_part = _sc_deg(dst3, n_chunks)
    degp = deg_part[:, :N].reshape(NC, N, 1)

    u1, dinv = _tc_scale_in(x_graph, W1, degp)
    s1 = _sc_edge(u1, src3, dst3, n_chunks)[:, :N, :]
    u2 = _tc_mid(s1, u1, dinv, b1.reshape(1, D_HID))
    s2 = _sc_edge(u2, src3, dst3, n_chunks)[:, :N, :]
    return _tc_out(s2, u2, dinv, W2, b2.reshape(1, D_OUT))
